# Initial kernel scaffold; baseline (speedup 1.0000x reference)
import jax
import jax.numpy as jnp
from jax.experimental import pallas as pl


def kernel(x, edge_index, W1, b1, W2, b2, W3, b3):
    raise NotImplementedError("write your pallas kernel here")


# speedup vs baseline: 21.1454x; 21.1454x over previous
import functools

import jax
import jax.numpy as jnp
from jax import lax
from jax.experimental import pallas as pl
from jax.experimental.pallas import tpu as pltpu
from jax.experimental.pallas import tpu_sc as plsc

N = 50000
E = 1600000
NP = 51200
NT = 16
NC = 2
NW = NC * NT
RPT = NP // NT
C = 128
EP = 391 * NW * C
EPT = EP // NW
NCHUNKS = EPT // C


def _make_agg(F):
    @functools.partial(
        pl.kernel,
        out_type=jax.ShapeDtypeStruct((NC * NP, F), jnp.float32),
        mesh=plsc.VectorSubcoreMesh(core_axis_name="c", subcore_axis_name="s"),
        compiler_params=pltpu.CompilerParams(use_tc_tiling_on_sc=False),
        scratch_types=[
            pltpu.VMEM_SHARED((NP, F), jnp.float32),
            pltpu.VMEM_SHARED((NP, F), jnp.float32),
            pltpu.VMEM((C,), jnp.int32),
            pltpu.VMEM((C,), jnp.int32),
            pltpu.VMEM((C, F), jnp.float32),
        ],
    )
    def agg(h_hbm, z_hbm, src_hbm, dst_hbm, out_hbm, stage, acc, srcv, dstv, rows):
        cid = lax.axis_index("c")
        sid = lax.axis_index("s")
        w = sid * NC + cid
        r0 = sid * RPT
        def stage_body(j, carry):
            r = r0 + j * C
            pltpu.sync_copy(h_hbm.at[pl.ds(r, C)], rows)
            pltpu.sync_copy(rows, stage.at[pl.ds(r, C)])
            pltpu.sync_copy(z_hbm.at[pl.ds(r, C)], rows)
            pltpu.sync_copy(rows, acc.at[pl.ds(r, C)])
            return carry

        lax.fori_loop(0, RPT // C, stage_body, 0)
        plsc.subcore_barrier()

        e0 = w * EPT

        def body(i, carry):
            base = e0 + i * C
            pltpu.sync_copy(src_hbm.at[pl.ds(base, C)], srcv)
            pltpu.sync_copy(dst_hbm.at[pl.ds(base, C)], dstv)
            pltpu.sync_copy(stage.at[srcv], rows)
            pltpu.sync_copy(rows, acc.at[dstv], add=True)
            return carry

        lax.fori_loop(0, NCHUNKS, body, 0)
        plsc.subcore_barrier()

        def out_body(j, carry):
            r = r0 + j * C
            pltpu.sync_copy(acc.at[pl.ds(r, C)], rows)
            pltpu.sync_copy(rows, out_hbm.at[pl.ds(cid * NP + r, C)])
            return carry

        lax.fori_loop(0, RPT // C, out_body, 0)

    return agg


_agg8 = _make_agg(8)
_agg16 = _make_agg(16)


_R = 1024


def _rows(F):
    return pl.BlockSpec((_R, F), lambda i: (i, 0))


def _full(shape):
    return pl.BlockSpec(shape, lambda i: tuple(0 for _ in shape))


def _tc(body, in_feats, full_shapes, out_feats):
    return pl.pallas_call(
        body,
        grid=(NP // _R,),
        in_specs=[_rows(f) for f in in_feats] + [_full(s) for s in full_shapes],
        out_specs=[_rows(f) for f in out_feats] if len(out_feats) > 1 else _rows(out_feats[0]),
        out_shape=(
            [jax.ShapeDtypeStruct((NP, f), jnp.float32) for f in out_feats]
            if len(out_feats) > 1
            else jax.ShapeDtypeStruct((NP, out_feats[0]), jnp.float32)
        ),
    )


def _pre_body(p0, p1, xp, d_out, xs_out):
    d = lax.rsqrt(p0[...][:, 0:1] + p1[...][:, 0:1] + 1.0)
    d_out[...] = d
    xs_out[...] = xp[...] * d


def _l1_body(s0, s1, xs, d, w1, b1, out):
    p = (s0[...] + s1[...] + xs[...]) * d[...]
    h = jnp.maximum(jnp.dot(p, w1[...], preferred_element_type=jnp.float32) + b1[...], 0.0)
    out[...] = h * d[...]


def _l2_body(s0, s1, h1, d, w2, b2, w3, out):
    p = (s0[...] + s1[...] + h1[...]) * d[...]
    h = jnp.maximum(jnp.dot(p, w2[...], preferred_element_type=jnp.float32) + b2[...], 0.0)
    out[...] = jnp.dot(h, w3[...], preferred_element_type=jnp.float32) * d[...]


def _l3_body(s0, s1, g, d, b3, out):
    out[...] = ((s0[...] + s1[...] + g[...]) * d[...])[:, 0:2] + b3[...]


_pre = _tc(_pre_body, [8, 8, 8], [], [1, 8])
_l1 = _tc(_l1_body, [8, 8, 8, 1], [(8, 16), (1, 16)], [16])
_l2 = _tc(_l2_body, [16, 16, 16, 1], [(16, 32), (1, 32), (32, 8)], [8])
_l3 = _tc(_l3_body, [8, 8, 8, 1], [(1, 2)], [2])


@jax.jit
def kernel(x, edge_index, W1, b1, W2, b2, W3, b3):
    ei = edge_index.astype(jnp.int32)
    pad = jnp.full((EP - E,), N, jnp.int32)
    src = jnp.concatenate([ei[0], pad])
    dst = jnp.concatenate([ei[1], pad])

    xp = jnp.zeros((NP, 8), jnp.float32).at[:N, :3].set(x)
    ones8 = jnp.ones((NP, 8), jnp.float32)
    z8 = jnp.zeros((NP, 8), jnp.float32)
    z16 = jnp.zeros((NP, 16), jnp.float32)
    w1p = jnp.zeros((8, 16), jnp.float32).at[:3].set(W1)
    w3p = jnp.zeros((32, 8), jnp.float32).at[:, :2].set(W3)

    degp = _agg8(ones8, z8, src, dst)
    d, xs = _pre(degp[:NP], degp[NP:], xp)

    s1 = _agg8(xs, z8, src, dst)
    h1p = _l1(s1[:NP], s1[NP:], xs, d, w1p, b1.reshape(1, 16))

    s2 = _agg16(h1p, z16, src, dst)
    gp = _l2(s2[:NP], s2[NP:], h1p, d, W2, b2.reshape(1, 32), w3p)

    s3 = _agg8(gp, z8, src, dst)
    out = _l3(s3[:NP], s3[NP:], gp, d, b3.reshape(1, 2))
    return out[:N]


# speedup vs baseline: 52.0258x; 2.4604x over previous
import functools

import jax
import jax.numpy as jnp
from jax import lax
from jax.experimental import pallas as pl
from jax.experimental.pallas import tpu as pltpu
from jax.experimental.pallas import tpu_sc as plsc

N = 50000
E = 1600000
NP = 51200
NT = 16
NC = 2
NW = NC * NT
RPT = NP // NT
C = 128
K = 8
NCHUNKS = 392
NSUPER = NCHUNKS // K
EP = NCHUNKS * NW * C
EPT = EP // NW


def _make_agg(F, gather=True):
    scratch = [
        pltpu.VMEM_SHARED((NP, F), jnp.float32),
        pltpu.VMEM((2, K, 2, C), jnp.int32),
        pltpu.VMEM((K, C, F), jnp.float32),
        pltpu.SemaphoreType.DMA((2,)),
        pltpu.SemaphoreType.DMA,
        pltpu.SemaphoreType.DMA,
    ]
    if gather:
        scratch = [pltpu.VMEM_SHARED((NP, F), jnp.float32)] + scratch

    @functools.partial(
        pl.kernel,
        out_type=jax.ShapeDtypeStruct((NC * NP, F), jnp.float32),
        mesh=plsc.VectorSubcoreMesh(core_axis_name="c", subcore_axis_name="s"),
        compiler_params=pltpu.CompilerParams(use_tc_tiling_on_sc=False),
        scratch_types=scratch,
    )
    def agg(h_hbm, z_hbm, epk_hbm, out_hbm, *refs):
        if gather:
            stage, acc, idx, rows, sem_i, sem_g, sem_s = refs
        else:
            acc, idx, rows, sem_i, sem_g, sem_s = refs
            stage = None
        cid = lax.axis_index("c")
        sid = lax.axis_index("s")
        w = sid * NC + cid
        r0 = sid * RPT

        def init_body(j, carry):
            r = r0 + j * C
            if gather:
                pltpu.sync_copy(h_hbm.at[pl.ds(r, C)], rows.at[0])
                pltpu.sync_copy(rows.at[0], stage.at[pl.ds(r, C)])
            pltpu.sync_copy(z_hbm.at[pl.ds(r, C)], rows.at[0])
            pltpu.sync_copy(rows.at[0], acc.at[pl.ds(r, C)])
            return carry

        lax.fori_loop(0, RPT // C, init_body, 0)
        if not gather:
            for k in range(K):
                pltpu.sync_copy(h_hbm.at[pl.ds(0, C)], rows.at[k])
        plsc.subcore_barrier()

        chunk0 = w * NCHUNKS
        for k in range(K):
            pltpu.async_copy(epk_hbm.at[chunk0 + k], idx.at[0, k], sem_i.at[0])

        def super_body(I, carry):
            b = lax.rem(I, 2)
            b1 = 1 - b

            @pl.when(I > 0)
            def _():
                for k in range(K):
                    pltpu.make_async_copy(
                        rows.at[k], acc.at[pl.ds(0, C)], sem_s
                    ).wait()

            @pl.when(I < NSUPER - 1)
            def _():
                base = chunk0 + (I + 1) * K
                for k in range(K):
                    pltpu.async_copy(
                        epk_hbm.at[base + k], idx.at[b1, k], sem_i.at[b1]
                    )

            for k in range(K):
                pltpu.make_async_copy(
                    epk_hbm.at[0], idx.at[b, k], sem_i.at[b]
                ).wait()

            if gather:
                descs = [
                    pltpu.async_copy(
                        stage.at[idx.at[b, k, 0]], rows.at[k], sem_g
                    )
                    for k in range(K)
                ]
                for k in range(K):
                    descs[k].wait()
                    pltpu.async_copy(
                        rows.at[k], acc.at[idx.at[b, k, 1]], sem_s, add=True
                    )
            else:
                for k in range(K):
                    pltpu.async_copy(
                        rows.at[k], acc.at[idx.at[b, k, 1]], sem_s, add=True
                    )
            return carry

        lax.fori_loop(0, NSUPER, super_body, 0)
        for k in range(K):
            pltpu.make_async_copy(rows.at[k], acc.at[pl.ds(0, C)], sem_s).wait()
        plsc.subcore_barrier()

        def out_body(j, carry):
            r = r0 + j * C
            pltpu.sync_copy(acc.at[pl.ds(r, C)], rows.at[0])
            pltpu.sync_copy(rows.at[0], out_hbm.at[pl.ds(cid * NP + r, C)])
            return carry

        lax.fori_loop(0, RPT // C, out_body, 0)

    return agg


_deg8 = _make_agg(8, gather=False)
_agg8 = _make_agg(8)
_agg16 = _make_agg(16)


_R = 1024


def _rows_spec(F):
    return pl.BlockSpec((_R, F), lambda i: (i, 0))


def _full(shape):
    return pl.BlockSpec(shape, lambda i: tuple(0 for _ in shape))


def _tc(body, in_feats, full_shapes, out_feats):
    return pl.pallas_call(
        body,
        grid=(NP // _R,),
        in_specs=[_rows_spec(f) for f in in_feats] + [_full(s) for s in full_shapes],
        out_specs=[_rows_spec(f) for f in out_feats] if len(out_feats) > 1 else _rows_spec(out_feats[0]),
        out_shape=(
            [jax.ShapeDtypeStruct((NP, f), jnp.float32) for f in out_feats]
            if len(out_feats) > 1
            else jax.ShapeDtypeStruct((NP, out_feats[0]), jnp.float32)
        ),
    )


def _pre_body(p0, p1, xp, d_out, xs_out):
    d = lax.rsqrt(p0[...][:, 0:1] + p1[...][:, 0:1] + 1.0)
    d_out[...] = d
    xs_out[...] = xp[...] * d


def _l1_body(s0, s1, xs, d, w1, b1, out):
    p = (s0[...] + s1[...] + xs[...]) * d[...]
    h = jnp.maximum(jnp.dot(p, w1[...], preferred_element_type=jnp.float32) + b1[...], 0.0)
    out[...] = h * d[...]


def _l2_body(s0, s1, h1, d, w2, b2, w3, out):
    p = (s0[...] + s1[...] + h1[...]) * d[...]
    h = jnp.maximum(jnp.dot(p, w2[...], preferred_element_type=jnp.float32) + b2[...], 0.0)
    out[...] = jnp.dot(h, w3[...], preferred_element_type=jnp.float32) * d[...]


def _l3_body(s0, s1, g, d, b3, out):
    out[...] = ((s0[...] + s1[...] + g[...]) * d[...])[:, 0:2] + b3[...]


_pre = _tc(_pre_body, [8, 8, 8], [], [1, 8])
_l1 = _tc(_l1_body, [8, 8, 8, 1], [(8, 16), (1, 16)], [16])
_l2 = _tc(_l2_body, [16, 16, 16, 1], [(16, 32), (1, 32), (32, 8)], [8])
_l3 = _tc(_l3_body, [8, 8, 8, 1], [(1, 2)], [2])


@jax.jit
def kernel(x, edge_index, W1, b1, W2, b2, W3, b3):
    ei = edge_index.astype(jnp.int32)
    pad = jnp.full((EP - E,), N, jnp.int32)
    src = jnp.concatenate([ei[0], pad]).reshape(EP // C, C)
    dst = jnp.concatenate([ei[1], pad]).reshape(EP // C, C)
    epk = jnp.stack([src, dst], axis=1)

    xp = jnp.zeros((NP, 8), jnp.float32).at[:N, :3].set(x)
    ones8 = jnp.ones((NP, 8), jnp.float32)
    z8 = jnp.zeros((NP, 8), jnp.float32)
    z16 = jnp.zeros((NP, 16), jnp.float32)
    w1p = jnp.zeros((8, 16), jnp.float32).at[:3].set(W1)
    w3p = jnp.zeros((32, 8), jnp.float32).at[:, :2].set(W3)

    degp = _deg8(ones8, z8, epk)
    d, xs = _pre(degp[:NP], degp[NP:], xp)

    s1 = _agg8(xs, z8, epk)
    h1p = _l1(s1[:NP], s1[NP:], xs, d, w1p, b1.reshape(1, 16))

    s2 = _agg16(h1p, z16, epk)
    gp = _l2(s2[:NP], s2[NP:], h1p, d, W2, b2.reshape(1, 32), w3p)

    s3 = _agg8(gp, z8, epk)
    out = _l3(s3[:NP], s3[NP:], gp, d, b3.reshape(1, 2))
    return out[:N]


# speedup vs baseline: 69.5656x; 1.3371x over previous
import functools

import jax
import jax.numpy as jnp
from jax import lax
from jax.experimental import pallas as pl
from jax.experimental.pallas import tpu as pltpu
from jax.experimental.pallas import tpu_sc as plsc

N = 50000
E = 1600000
NP = 51200
NT = 16
NC = 2
NW = NC * NT
RPT = NP // NT
C = 128
K = 8
NCHUNKS = 392
NSUPER = NCHUNKS // K
EP = NCHUNKS * NW * C
NCK = EP // C


def _make_agg(F, gather=True):
    scratch = [
        pltpu.VMEM_SHARED((NP, F), jnp.float32),
        pltpu.VMEM((2, K, 2, C), jnp.int32),
        pltpu.VMEM((K, C, F), jnp.float32),
        pltpu.SemaphoreType.DMA((2,)),
        pltpu.SemaphoreType.DMA,
        pltpu.SemaphoreType.DMA,
    ]
    if gather:
        scratch = [pltpu.VMEM_SHARED((NP, F), jnp.float32)] + scratch
    sides = (0, 1) if gather else (1,)

    @functools.partial(
        pl.kernel,
        out_type=jax.ShapeDtypeStruct((NC * NP, F), jnp.float32),
        mesh=plsc.VectorSubcoreMesh(core_axis_name="c", subcore_axis_name="s"),
        compiler_params=pltpu.CompilerParams(use_tc_tiling_on_sc=False),
        scratch_types=scratch,
    )
    def agg(h_hbm, z_hbm, epk_hbm, out_hbm, *refs):
        if gather:
            stage, acc, idx, rows, sem_i, sem_g, sem_s = refs
        else:
            acc, idx, rows, sem_i, sem_g, sem_s = refs
            stage = None
        cid = lax.axis_index("c")
        sid = lax.axis_index("s")
        w = sid * NC + cid
        r0 = sid * RPT

        def init_body(j, carry):
            r = r0 + j * C
            if gather:
                pltpu.sync_copy(h_hbm.at[pl.ds(r, C)], rows.at[0])
                pltpu.sync_copy(rows.at[0], stage.at[pl.ds(r, C)])
            pltpu.sync_copy(z_hbm.at[pl.ds(r, C)], rows.at[0])
            pltpu.sync_copy(rows.at[0], acc.at[pl.ds(r, C)])
            return carry

        lax.fori_loop(0, RPT // C, init_body, 0)
        if not gather:
            for k in range(K):
                pltpu.sync_copy(h_hbm.at[pl.ds(0, C)], rows.at[k])
        plsc.subcore_barrier()

        chunk0 = w * NCHUNKS
        for k in range(K):
            for s in sides:
                pltpu.async_copy(
                    epk_hbm.at[s, chunk0 + k], idx.at[0, k, s], sem_i.at[0]
                )

        def super_body(I, carry):
            b = lax.rem(I, 2)
            b1 = 1 - b

            @pl.when(I > 0)
            def _():
                for k in range(K):
                    pltpu.make_async_copy(
                        rows.at[k], acc.at[pl.ds(0, C)], sem_s
                    ).wait()

            @pl.when(I < NSUPER - 1)
            def _():
                base = chunk0 + (I + 1) * K
                for k in range(K):
                    for s in sides:
                        pltpu.async_copy(
                            epk_hbm.at[s, base + k], idx.at[b1, k, s],
                            sem_i.at[b1],
                        )

            for k in range(K):
                for s in sides:
                    pltpu.make_async_copy(
                        epk_hbm.at[0, 0], idx.at[b, k, s], sem_i.at[b]
                    ).wait()

            if gather:
                descs = [
                    pltpu.async_copy(
                        stage.at[idx.at[b, k, 0]], rows.at[k], sem_g
                    )
                    for k in range(K)
                ]
                for k in range(K):
                    descs[k].wait()
                    pltpu.async_copy(
                        rows.at[k], acc.at[idx.at[b, k, 1]], sem_s, add=True
                    )
            else:
                for k in range(K):
                    pltpu.async_copy(
                        rows.at[k], acc.at[idx.at[b, k, 1]], sem_s, add=True
                    )
            return carry

        lax.fori_loop(0, NSUPER, super_body, 0)
        for k in range(K):
            pltpu.make_async_copy(rows.at[k], acc.at[pl.ds(0, C)], sem_s).wait()
        plsc.subcore_barrier()

        def out_body(j, carry):
            r = r0 + j * C
            pltpu.sync_copy(acc.at[pl.ds(r, C)], rows.at[0])
            pltpu.sync_copy(rows.at[0], out_hbm.at[pl.ds(cid * NP + r, C)])
            return carry

        lax.fori_loop(0, RPT // C, out_body, 0)

    return agg


_deg8 = _make_agg(8, gather=False)
_agg8 = _make_agg(8)
_agg16 = _make_agg(16)


_R = 6400
_G = NP // _R


def _rows_spec(F):
    return pl.BlockSpec((_R, F), lambda i: (i, 0))


def _half_specs(F):
    return [
        pl.BlockSpec((_R, F), lambda i: (i, 0)),
        pl.BlockSpec((_R, F), lambda i: (i + _G, 0)),
    ]


def _full(shape):
    return pl.BlockSpec(shape, lambda i: tuple(0 for _ in shape))


def _tc(body, in_specs, out_feats, out_shapes=None):
    if out_shapes is None:
        out_shapes = [(NP, f) for f in out_feats]
    return pl.pallas_call(
        body,
        grid=(_G,),
        in_specs=in_specs,
        out_specs=(
            [_rows_spec(f) for f in out_feats]
            if len(out_feats) > 1
            else _rows_spec(out_feats[0])
        ),
        out_shape=(
            [jax.ShapeDtypeStruct(s, jnp.float32) for s in out_shapes]
            if len(out_shapes) > 1
            else jax.ShapeDtypeStruct(out_shapes[0], jnp.float32)
        ),
    )


def _pre_body(p0, p1, x, d_out, xs_out):
    i = pl.program_id(0)
    row = lax.broadcasted_iota(jnp.int32, (_R, 1), 0) + i * _R
    valid = row < N
    d = lax.rsqrt(p0[...][:, 0:1] + p1[...][:, 0:1] + 1.0)
    d_out[...] = d
    xw = jnp.where(valid, x[...], 0.0) * d
    xs_out[...] = jnp.pad(xw, ((0, 0), (0, 5)))


def _l1_body(s0, s1, xs, d, w1, b1, out):
    p = (s0[...] + s1[...] + xs[...]) * d[...]
    h = jnp.maximum(jnp.dot(p, w1[...], preferred_element_type=jnp.float32) + b1[...], 0.0)
    out[...] = h * d[...]


def _l2_body(s0, s1, h1, d, w2, b2, w3, out):
    p = (s0[...] + s1[...] + h1[...]) * d[...]
    h = jnp.maximum(jnp.dot(p, w2[...], preferred_element_type=jnp.float32) + b2[...], 0.0)
    out[...] = jnp.dot(h, w3[...], preferred_element_type=jnp.float32) * d[...]


def _l3_body(s0, s1, g, d, b3, out):
    out[...] = ((s0[...] + s1[...] + g[...]) * d[...])[:, 0:2] + b3[...]


_pre = _tc(
    _pre_body,
    _half_specs(8) + [pl.BlockSpec((_R, 3), lambda i: (i, 0))],
    [1, 8],
)
_l1 = _tc(
    _l1_body,
    _half_specs(8) + [_rows_spec(8), _rows_spec(1), _full((8, 16)), _full((1, 16))],
    [16],
)
_l2 = _tc(
    _l2_body,
    _half_specs(16)
    + [_rows_spec(16), _rows_spec(1), _full((16, 32)), _full((1, 32)), _full((32, 8))],
    [8],
)
_l3 = _tc(
    _l3_body,
    _half_specs(8) + [_rows_spec(8), _rows_spec(1), _full((1, 2))],
    [2],
    out_shapes=[(N, 2)],
)


@jax.jit
def kernel(x, edge_index, W1, b1, W2, b2, W3, b3):
    ei = edge_index.astype(jnp.int32)
    padk = jnp.full((2, EP - E), N, jnp.int32)
    epk = jnp.concatenate([ei, padk], axis=1).reshape(2, NCK, C)

    ones8 = jnp.ones((NP, 8), jnp.float32)
    z8 = jnp.zeros((NP, 8), jnp.float32)
    z16 = jnp.zeros((NP, 16), jnp.float32)
    w1p = jnp.zeros((8, 16), jnp.float32).at[:3].set(W1)
    w3p = jnp.zeros((32, 8), jnp.float32).at[:, :2].set(W3)

    degp = _deg8(ones8, z8, epk)
    d, xs = _pre(degp, degp, x)

    s1 = _agg8(xs, z8, epk)
    h1p = _l1(s1, s1, xs, d, w1p, b1.reshape(1, 16))

    s2 = _agg16(h1p, z16, epk)
    gp = _l2(s2, s2, h1p, d, W2, b2.reshape(1, 32), w3p)

    s3 = _agg8(gp, z8, epk)
    return _l3(s3, s3, gp, d, b3.reshape(1, 2))
